# Initial kernel scaffold; baseline (speedup 1.0000x reference)
import jax
import jax.numpy as jnp
from jax.experimental import pallas as pl


def kernel(ast_nodes_encodings, ast_nodes_with_symbol_leaf_nodes_indices, ast_nodes_with_symbol_leaf_symbol_idx):
    raise NotImplementedError("write your pallas kernel here")


# speedup vs baseline: 1.5446x; 1.5446x over previous
import functools

import jax
import jax.numpy as jnp
from jax import lax
from jax.experimental import pallas as pl
from jax.experimental.pallas import tpu as pltpu
from jax.experimental.pallas import tpu_sc as plsc

N_NODES_ = 16384
D_ = 1024
N_OCC_ = 8192

_info = plsc.get_sparse_core_info()
_NC, _NS = _info.num_cores, _info.num_subcores
_NW = _NC * _NS
_BPW = N_OCC_ // _NW
_CHUNK = 32
_NCHUNK = _BPW // _CHUNK
_NBUF = 3


def _gather_body(table_hbm, idx_hbm, out_hbm, idx_v, bufs, gsems, ssems):
    wid = lax.axis_index("s") * _NC + lax.axis_index("c")
    base = wid * _BPW
    pltpu.sync_copy(idx_hbm.at[wid], idx_v)

    gops = [None] * _NBUF
    for b in range(_NBUF):
        gops[b] = pltpu.async_copy(table_hbm.at[idx_v.at[b]], bufs[b], gsems[b])
    last_store = [None] * _NBUF
    for c in range(_NCHUNK):
        b = c % _NBUF
        gops[b].wait()
        sop = pltpu.async_copy(
            bufs[b], out_hbm.at[pl.ds(base + c * _CHUNK, _CHUNK)], ssems[b])
        nxt = c + _NBUF
        if nxt < _NCHUNK:
            sop.wait()
            gops[b] = pltpu.async_copy(
                table_hbm.at[idx_v.at[nxt]], bufs[b], gsems[b])
        else:
            last_store[b] = sop
    for b in range(_NBUF):
        last_store[b].wait()


def _body(table_hbm, idx_hbm, out_hbm, idx_v, b0, b1, b2, g0, g1, g2,
          s0, s1, s2):
    _gather_body(table_hbm, idx_hbm, out_hbm, idx_v,
                 [b0, b1, b2], [g0, g1, g2], [s0, s1, s2])


@jax.jit
def _gather(table, idx):
    mesh = plsc.VectorSubcoreMesh(core_axis_name="c", subcore_axis_name="s")
    idx3 = idx.reshape(_NW, _NCHUNK, _CHUNK)
    run = pl.kernel(
        _body,
        mesh=mesh,
        out_type=jax.ShapeDtypeStruct((N_OCC_, D_), jnp.float32),
        scratch_types=[
            pltpu.VMEM((_NCHUNK, _CHUNK), jnp.int32),
            pltpu.VMEM((_CHUNK, D_), jnp.float32),
            pltpu.VMEM((_CHUNK, D_), jnp.float32),
            pltpu.VMEM((_CHUNK, D_), jnp.float32),
            pltpu.SemaphoreType.DMA,
            pltpu.SemaphoreType.DMA,
            pltpu.SemaphoreType.DMA,
            pltpu.SemaphoreType.DMA,
            pltpu.SemaphoreType.DMA,
            pltpu.SemaphoreType.DMA,
        ],
    )
    return run(table, idx3)


def kernel(ast_nodes_encodings, ast_nodes_with_symbol_leaf_nodes_indices,
           ast_nodes_with_symbol_leaf_symbol_idx):
    out = _gather(ast_nodes_encodings, ast_nodes_with_symbol_leaf_nodes_indices)
    return (out, ast_nodes_with_symbol_leaf_symbol_idx)
